# baseline (speedup 1.0000x reference)
import functools

import jax
import jax.numpy as jnp
from jax import lax
from jax.experimental import pallas as pl
from jax.experimental.pallas import tpu as pltpu
from jax.experimental.pallas import tpu_sc as plsc

_B = 4096
_S = 200
_D = 64
_H = 256
_C = 10
_NC = 2
_NS = 16
_NW = _NC * _NS
_RPT = _B // _NW
_HS = _S // 2
_PW = 16


def _pool_body(seq_hbm, ng_hbm, pbw_hbm, pbn_hbm, wtab_hbm, ntab_hbm, out_hbm,
               idxw_v, idxn_v, pbw_v, pbn_v, bufs, acc, sem0, sem1):
    wid = lax.axis_index("c") * _NS + lax.axis_index("s")
    base = wid * _RPT

    pltpu.sync_copy(pbw_hbm.at[pl.ds(base, _RPT)], pbw_v)
    pltpu.sync_copy(pbn_hbm.at[pl.ds(base, _RPT)], pbn_v)

    sems = (sem0, sem1)
    idxs = (idxw_v, idxn_v)
    tabs = (wtab_hbm, ntab_hbm)
    pbs = (pbw_v, pbn_v)

    def stage_idx(h):
        pltpu.sync_copy(seq_hbm.at[pl.ds(2 * base + 128 * h, 128)], idxw_v)
        pltpu.sync_copy(ng_hbm.at[pl.ds(2 * base + 128 * h, 128)], idxn_v)

    def fire(rloc, t):
        j = 2 * rloc
        pltpu.async_copy(tabs[t].at[idxs[t].at[j]], bufs.at[2 * t + 0], sems[t])
        pltpu.async_copy(tabs[t].at[idxs[t].at[j + 1]], bufs.at[2 * t + 1], sems[t])

    def drain(rloc, t):
        j = 2 * rloc
        pltpu.make_async_copy(tabs[t].at[idxs[t].at[j]], bufs.at[2 * t + 0], sems[t]).wait()
        pltpu.make_async_copy(tabs[t].at[idxs[t].at[j + 1]], bufs.at[2 * t + 1], sems[t]).wait()

    def reduce_half(row, accrow, t):
        pbvec = pbs[t][row, pl.ds(0, _PW)]
        accs = tuple(jnp.zeros((16,), jnp.float32) for _ in range(4))
        for c01 in range(2):
            t0 = _HS * c01
            bslot = 2 * t + c01
            lo = t0
            while lo < t0 + _HS:
                widx = lo >> 5
                hi = min(32 * (widx + 1), t0 + _HS)
                word = pbvec[widx]

                def body(i, carry, word=word, t0=t0, bslot=bslot):
                    hb = ((word >> (i & 31)) & 1) << 6
                    return tuple(
                        carry[k] + bufs[bslot, i - t0, pl.ds(hb + 16 * k, 16)]
                        for k in range(4))

                accs = lax.fori_loop(lo, hi, body, accs)
                lo = hi
        for k in range(4):
            acc[accrow, pl.ds(64 * t + 16 * k, 16)] = accs[k]

    stage_idx(0)
    fire(0, 0)
    fire(0, 1)
    for h in range(2):
        for g in range(2):
            last = g == 1

            def grp(r, carry, h=h, g=g, last=last):
                rloc = 32 * g + r
                for t in range(2):
                    drain(rloc, t)
                    reduce_half(64 * h + rloc, r, t)
                    fire(rloc + 1, t)
                return carry

            lax.fori_loop(0, 31 if last else 32, grp, 0)
            if last:
                for t in range(2):
                    drain(63, t)
                    reduce_half(64 * h + 63, 31, t)
            pltpu.sync_copy(
                acc, out_hbm.at[pl.ds(base + 64 * h + 32 * g, 32)])
            if last and h == 0:
                stage_idx(1)
                fire(0, 0)
                fire(0, 1)


@functools.partial(
    pl.kernel,
    mesh=plsc.VectorSubcoreMesh(core_axis_name="c", subcore_axis_name="s"),
    out_type=jax.ShapeDtypeStruct((_B, 2 * _D), jnp.float32),
    scratch_types=[
        pltpu.VMEM((_RPT, _HS), jnp.int32),
        pltpu.VMEM((_RPT, _HS), jnp.int32),
        pltpu.VMEM((_RPT, _PW), jnp.int32),
        pltpu.VMEM((_RPT, _PW), jnp.int32),
        pltpu.VMEM((4, _HS, 2 * _D), jnp.float32),
        pltpu.VMEM((32, 2 * _D), jnp.float32),
        pltpu.SemaphoreType.DMA,
        pltpu.SemaphoreType.DMA,
    ],
)
def _pool(seq_hbm, ng_hbm, pbw_hbm, pbn_hbm, wtab_hbm, ntab_hbm, out_hbm,
          idxw_v, idxn_v, pbw_v, pbn_v, bufs, acc, sem0, sem1):
    _pool_body(seq_hbm, ng_hbm, pbw_hbm, pbn_hbm, wtab_hbm, ntab_hbm, out_hbm,
               idxw_v, idxn_v, pbw_v, pbn_v, bufs, acc, sem0, sem1)


_BM = 512


def _mlp_body(x_ref, w1_ref, b1_ref, w2_ref, b2_ref, o_ref):
    x = x_ref[...] * (1.0 / _S)
    h = lax.dot_general(x, w1_ref[...], (((1,), (1,)), ((), ())),
                        preferred_element_type=jnp.float32)
    h = jnp.maximum(h + b1_ref[...], 0.0)
    o = lax.dot_general(h, w2_ref[...], (((1,), (1,)), ((), ())),
                        preferred_element_type=jnp.float32)
    o_ref[...] = o + b2_ref[...]


def _mlp(xsum, W1, b1, W2, b2):
    return pl.pallas_call(
        _mlp_body,
        grid=(_B // _BM,),
        in_specs=[
            pl.BlockSpec((_BM, 2 * _D), lambda i: (i, 0)),
            pl.BlockSpec((_H, 2 * _D), lambda i: (0, 0)),
            pl.BlockSpec((1, _H), lambda i: (0, 0)),
            pl.BlockSpec((_C, _H), lambda i: (0, 0)),
            pl.BlockSpec((1, _C), lambda i: (0, 0)),
        ],
        out_specs=pl.BlockSpec((_BM, _C), lambda i: (i, 0)),
        out_shape=jax.ShapeDtypeStruct((_B, _C), jnp.float32),
    )(xsum, W1, b1.reshape(1, _H), W2, b2.reshape(1, _C))


def _packbits(idx):
    par = (idx & 1).astype(jnp.uint32)
    par = jnp.pad(par, ((0, 0), (0, 32 * _PW - _S)))
    par = par.reshape(_B, _PW, 32)
    shifts = jnp.arange(32, dtype=jnp.uint32)[None, None, :]
    return (par << shifts).sum(axis=-1).astype(jnp.int32)


def kernel(sequence, ngrams, word_emb, ngram_emb, W1, b1, W2, b2):
    seq = sequence.astype(jnp.int32)
    ng = ngrams.astype(jnp.int32)
    seqh = (seq >> 1).reshape(2 * _B, _HS)
    ngh = (ng >> 1).reshape(2 * _B, _HS)
    pbw = _packbits(seq)
    pbn = _packbits(ng)
    wtab = word_emb.reshape(-1, 2 * _D)
    ntab = ngram_emb.reshape(-1, 2 * _D)
    xsum = _pool(seqh, ngh, pbw, pbn, wtab, ntab)
    return _mlp(xsum, W1, b1, W2, b2)


# speedup vs baseline: 1.1205x; 1.1205x over previous
import functools

import jax
import jax.numpy as jnp
from jax import lax
from jax.experimental import pallas as pl
from jax.experimental.pallas import tpu as pltpu
from jax.experimental.pallas import tpu_sc as plsc

_B = 4096
_S = 200
_D = 64
_H = 256
_C = 10
_NC = 2
_NS = 16
_NW = _NC * _NS
_RPT = _B // _NW
_HS = _S // 2

_TB = 2048


def _trans_body(in_ref, o_ref):
    xt = jnp.transpose(in_ref[...], (1, 0))
    o_ref[...] = jnp.concatenate(
        [xt, jnp.zeros((_TB, _D), jnp.float32)], axis=1)


def _transpose_table(tabT, n_rows):
    grid = (n_rows + _TB - 1) // _TB
    return pl.pallas_call(
        _trans_body,
        grid=(grid,),
        in_specs=[pl.BlockSpec((_D, _TB), lambda i: (0, i))],
        out_specs=pl.BlockSpec((_TB, 2 * _D), lambda i: (i, 0)),
        out_shape=jax.ShapeDtypeStruct((n_rows, 2 * _D), jnp.float32),
    )(tabT)


def _pool_body(seq_hbm, ng_hbm, wtab_hbm, ntab_hbm, out_hbm,
               idxw_v, idxn_v, bufs, acc, sem0, sem1):
    wid = lax.axis_index("c") * _NS + lax.axis_index("s")
    base = wid * _RPT

    sems = (sem0, sem1)
    idxs = (idxw_v, idxn_v)
    tabs = (wtab_hbm, ntab_hbm)

    def stage_idx(h):
        pltpu.sync_copy(seq_hbm.at[pl.ds(2 * base + 128 * h, 128)], idxw_v)
        pltpu.sync_copy(ng_hbm.at[pl.ds(2 * base + 128 * h, 128)], idxn_v)

    def fire(rloc, t):
        j = 2 * rloc
        pltpu.async_copy(tabs[t].at[idxs[t].at[j]], bufs.at[2 * t + 0], sems[t])
        pltpu.async_copy(tabs[t].at[idxs[t].at[j + 1]], bufs.at[2 * t + 1], sems[t])

    def drain(rloc, t):
        j = 2 * rloc
        pltpu.make_async_copy(tabs[t].at[idxs[t].at[j]], bufs.at[2 * t + 0], sems[t]).wait()
        pltpu.make_async_copy(tabs[t].at[idxs[t].at[j + 1]], bufs.at[2 * t + 1], sems[t]).wait()

    def reduce_half(accrow, t):
        accs = tuple(jnp.zeros((16,), jnp.float32) for _ in range(4))
        for c01 in range(2):
            bslot = 2 * t + c01

            def body(i, carry, bslot=bslot):
                return tuple(
                    carry[k] + bufs[bslot, i, pl.ds(16 * k, 16)]
                    for k in range(4))

            accs = lax.fori_loop(0, _HS, body, accs)
        for k in range(4):
            acc[accrow, pl.ds(64 * t + 16 * k, 16)] = accs[k]

    stage_idx(0)
    fire(0, 0)
    fire(0, 1)
    for h in range(2):
        for g in range(2):
            last = g == 1

            def grp(r, carry, g=g):
                rloc = 32 * g + r
                for t in range(2):
                    drain(rloc, t)
                    reduce_half(r, t)
                    fire(rloc + 1, t)
                return carry

            lax.fori_loop(0, 31 if last else 32, grp, 0)
            if last:
                for t in range(2):
                    drain(63, t)
                    reduce_half(31, t)
            pltpu.sync_copy(
                acc, out_hbm.at[pl.ds(base + 64 * h + 32 * g, 32)])
            if last and h == 0:
                stage_idx(1)
                fire(0, 0)
                fire(0, 1)


@functools.partial(
    pl.kernel,
    mesh=plsc.VectorSubcoreMesh(core_axis_name="c", subcore_axis_name="s"),
    out_type=jax.ShapeDtypeStruct((_B, 2 * _D), jnp.float32),
    scratch_types=[
        pltpu.VMEM((_RPT, _HS), jnp.int32),
        pltpu.VMEM((_RPT, _HS), jnp.int32),
        pltpu.VMEM((4, _HS, 2 * _D), jnp.float32),
        pltpu.VMEM((32, 2 * _D), jnp.float32),
        pltpu.SemaphoreType.DMA,
        pltpu.SemaphoreType.DMA,
    ],
)
def _pool(seq_hbm, ng_hbm, wtab_hbm, ntab_hbm, out_hbm,
          idxw_v, idxn_v, bufs, acc, sem0, sem1):
    _pool_body(seq_hbm, ng_hbm, wtab_hbm, ntab_hbm, out_hbm,
               idxw_v, idxn_v, bufs, acc, sem0, sem1)


_BM = 512


def _mlp_body(x_ref, w1_ref, b1_ref, w2_ref, b2_ref, o_ref):
    x = x_ref[...] * (1.0 / _S)
    h = lax.dot_general(x, w1_ref[...], (((1,), (1,)), ((), ())),
                        preferred_element_type=jnp.float32)
    h = jnp.maximum(h + b1_ref[...], 0.0)
    o = lax.dot_general(h, w2_ref[...], (((1,), (1,)), ((), ())),
                        preferred_element_type=jnp.float32)
    o_ref[...] = o + b2_ref[...]


def _mlp(xsum, W1, b1, W2, b2):
    return pl.pallas_call(
        _mlp_body,
        grid=(_B // _BM,),
        in_specs=[
            pl.BlockSpec((_BM, 2 * _D), lambda i: (i, 0)),
            pl.BlockSpec((_H, 2 * _D), lambda i: (0, 0)),
            pl.BlockSpec((1, _H), lambda i: (0, 0)),
            pl.BlockSpec((_C, _H), lambda i: (0, 0)),
            pl.BlockSpec((1, _C), lambda i: (0, 0)),
        ],
        out_specs=pl.BlockSpec((_BM, _C), lambda i: (i, 0)),
        out_shape=jax.ShapeDtypeStruct((_B, _C), jnp.float32),
    )(xsum, W1, b1.reshape(1, _H), W2, b2.reshape(1, _C))


def kernel(sequence, ngrams, word_emb, ngram_emb, W1, b1, W2, b2):
    seq2 = sequence.astype(jnp.int32).reshape(2 * _B, _HS)
    ng2 = ngrams.astype(jnp.int32).reshape(2 * _B, _HS)
    wtab = _transpose_table(word_emb.T, word_emb.shape[0])
    ntab = _transpose_table(ngram_emb.T, ngram_emb.shape[0])
    xsum = _pool(seq2, ng2, wtab, ntab)
    return _mlp(xsum, W1, b1, W2, b2)


# speedup vs baseline: 1.1917x; 1.0636x over previous
import functools

import jax
import jax.numpy as jnp
from jax import lax
from jax.experimental import pallas as pl
from jax.experimental.pallas import tpu as pltpu
from jax.experimental.pallas import tpu_sc as plsc

_B = 4096
_S = 200
_D = 64
_H = 256
_C = 10
_NC = 2
_NS = 16
_NW = _NC * _NS
_RPT = _B // _NW
_HS = _S // 2
_PW = 16

_TB = 2048


def _trans_body(in_ref, o_ref):
    xt = jnp.transpose(in_ref[...], (1, 0))
    o_ref[...] = jnp.concatenate(
        [xt[: _TB // 2], xt[_TB // 2:]], axis=1)


def _transpose_table(tabT, n_rows):
    grid = (n_rows + _TB - 1) // _TB
    return pl.pallas_call(
        _trans_body,
        grid=(grid,),
        in_specs=[pl.BlockSpec((_D, _TB), lambda i: (0, i))],
        out_specs=pl.BlockSpec((_TB // 2, 2 * _D), lambda i: (i, 0)),
        out_shape=jax.ShapeDtypeStruct((grid * _TB // 2, 2 * _D), jnp.float32),
    )(tabT)


def _fold_idx(idx):
    jh = ((idx >> 11) << 10) | (idx & 1023)
    hb = (idx >> 10) & 1
    return jh, hb


def _pool_body(idx_hbm, pb_hbm, tab_hbm, out_hbm, idx_v, pb_v, bufs, acc, sem0, sem1):
    wid = lax.axis_index("c") * _NS + lax.axis_index("s")
    base = wid * _RPT

    pltpu.sync_copy(idx_hbm.at[pl.ds(2 * base, 2 * _RPT)], idx_v)
    pltpu.sync_copy(pb_hbm.at[pl.ds(base, _RPT)], pb_v)
    sems = (sem0, sem1)

    def fire(row, s):
        j = 2 * row
        pltpu.async_copy(tab_hbm.at[idx_v.at[j]], bufs.at[2 * s + 0], sems[s])
        pltpu.async_copy(tab_hbm.at[idx_v.at[j + 1]], bufs.at[2 * s + 1], sems[s])

    def drain(row, s):
        j = 2 * row
        pltpu.make_async_copy(tab_hbm.at[idx_v.at[j]], bufs.at[2 * s + 0], sems[s]).wait()
        pltpu.make_async_copy(tab_hbm.at[idx_v.at[j + 1]], bufs.at[2 * s + 1], sems[s]).wait()

    def reduce_row(row, s):
        pbvec = pb_v[row, pl.ds(0, _PW)]
        accs = tuple(jnp.zeros((16,), jnp.float32) for _ in range(4))
        for c01 in range(2):
            t0 = _HS * c01
            bslot = 2 * s + c01
            lo = t0
            while lo < t0 + _HS:
                widx = lo >> 5
                hi = min(32 * (widx + 1), t0 + _HS)
                word = pbvec[widx]

                def body(i, carry, word=word, t0=t0, bslot=bslot):
                    hb = ((word >> (i & 31)) & 1) << 6
                    return tuple(
                        carry[k] + bufs[bslot, i - t0, pl.ds(hb + 16 * k, 16)]
                        for k in range(4))

                accs = lax.fori_loop(lo, hi, body, accs)
                lo = hi
        for k in range(4):
            acc[row, pl.ds(16 * k, 16)] = accs[k]

    fire(0, 0)
    fire(1, 1)

    def outer(rr, carry):
        for s in range(2):
            row = 2 * rr + s
            drain(row, s)
            reduce_row(row, s)
            fire(row + 2, s)
        return carry

    lax.fori_loop(0, _RPT // 2 - 1, outer, 0)
    for s in range(2):
        drain(_RPT - 2 + s, s)
        reduce_row(_RPT - 2 + s, s)

    pltpu.sync_copy(acc, out_hbm.at[pl.ds(base, _RPT)])


@functools.partial(
    pl.kernel,
    mesh=plsc.VectorSubcoreMesh(core_axis_name="c", subcore_axis_name="s"),
    out_type=jax.ShapeDtypeStruct((_B, _D), jnp.float32),
    scratch_types=[
        pltpu.VMEM((2 * _RPT, _HS), jnp.int32),
        pltpu.VMEM((_RPT, _PW), jnp.int32),
        pltpu.VMEM((4, _HS, 2 * _D), jnp.float32),
        pltpu.VMEM((_RPT, _D), jnp.float32),
        pltpu.SemaphoreType.DMA,
        pltpu.SemaphoreType.DMA,
    ],
)
def _pool1(idx_hbm, pb_hbm, tab_hbm, out_hbm, idx_v, pb_v, bufs, acc, sem0, sem1):
    _pool_body(idx_hbm, pb_hbm, tab_hbm, out_hbm, idx_v, pb_v, bufs, acc, sem0, sem1)


_BM = 512


def _mlp_body(xw_ref, xn_ref, w1_ref, b1_ref, w2_ref, b2_ref, o_ref):
    x = jnp.concatenate([xw_ref[...], xn_ref[...]], axis=1) * (1.0 / _S)
    h = lax.dot_general(x, w1_ref[...], (((1,), (1,)), ((), ())),
                        preferred_element_type=jnp.float32)
    h = jnp.maximum(h + b1_ref[...], 0.0)
    o = lax.dot_general(h, w2_ref[...], (((1,), (1,)), ((), ())),
                        preferred_element_type=jnp.float32)
    o_ref[...] = o + b2_ref[...]


def _mlp(xw, xn, W1, b1, W2, b2):
    return pl.pallas_call(
        _mlp_body,
        grid=(_B // _BM,),
        in_specs=[
            pl.BlockSpec((_BM, _D), lambda i: (i, 0)),
            pl.BlockSpec((_BM, _D), lambda i: (i, 0)),
            pl.BlockSpec((_H, 2 * _D), lambda i: (0, 0)),
            pl.BlockSpec((1, _H), lambda i: (0, 0)),
            pl.BlockSpec((_C, _H), lambda i: (0, 0)),
            pl.BlockSpec((1, _C), lambda i: (0, 0)),
        ],
        out_specs=pl.BlockSpec((_BM, _C), lambda i: (i, 0)),
        out_shape=jax.ShapeDtypeStruct((_B, _C), jnp.float32),
    )(xw, xn, W1, b1.reshape(1, _H), W2, b2.reshape(1, _C))


def _packbits(hb):
    par = hb.astype(jnp.uint32)
    par = jnp.pad(par, ((0, 0), (0, 32 * _PW - _S)))
    par = par.reshape(_B, _PW, 32)
    shifts = jnp.arange(32, dtype=jnp.uint32)[None, None, :]
    return (par << shifts).sum(axis=-1).astype(jnp.int32)


def kernel(sequence, ngrams, word_emb, ngram_emb, W1, b1, W2, b2):
    seq_jh, seq_hb = _fold_idx(sequence.astype(jnp.int32))
    ng_jh, ng_hb = _fold_idx(ngrams.astype(jnp.int32))
    seqh = seq_jh.reshape(2 * _B, _HS)
    ngh = ng_jh.reshape(2 * _B, _HS)
    pbw = _packbits(seq_hb)
    pbn = _packbits(ng_hb)
    wtab = _transpose_table(word_emb.T, word_emb.shape[0])
    xw = _pool1(seqh, pbw, wtab)
    ntab = _transpose_table(ngram_emb.T, ngram_emb.shape[0])
    xn = _pool1(ngh, pbn, ntab)
    return _mlp(xw, xn, W1, b1, W2, b2)


# speedup vs baseline: 1.5729x; 1.3199x over previous
import functools

import jax
import jax.numpy as jnp
from jax import lax
from jax.experimental import pallas as pl
from jax.experimental.pallas import tpu as pltpu
from jax.experimental.pallas import tpu_sc as plsc

_B = 4096
_S = 200
_D = 64
_H = 256
_C = 10
_NC = 2
_NS = 16
_NW = _NC * _NS
_RPT = _B // _NW
_HS = _S // 2
_PW = 16

_TB = 16384


def _trans_body(in_ref, o_ref):
    xt = jnp.transpose(in_ref[...], (1, 0))
    o_ref[...] = jnp.concatenate(
        [xt[: _TB // 2], xt[_TB // 2:]], axis=1)


def _transpose_table(tabT, n_rows):
    grid = (n_rows + _TB - 1) // _TB
    return pl.pallas_call(
        _trans_body,
        grid=(grid,),
        in_specs=[pl.BlockSpec((_D, _TB), lambda i: (0, i))],
        out_specs=pl.BlockSpec((_TB // 2, 2 * _D), lambda i: (i, 0)),
        out_shape=jax.ShapeDtypeStruct((grid * _TB // 2, 2 * _D), jnp.float32),
    )(tabT)


def _fold_idx(idx):
    half = _TB // 2
    jh = (idx // _TB) * half + (idx % half)
    hb = (idx % _TB) // half
    return jh, hb


def _pool_body(idx_hbm, pb_hbm, tab_hbm, out_hbm, idx_v, pb_v, bufs, acc, sem0, sem1):
    wid = lax.axis_index("c") * _NS + lax.axis_index("s")
    base = wid * _RPT

    pltpu.sync_copy(idx_hbm.at[pl.ds(2 * base, 2 * _RPT)], idx_v)
    pltpu.sync_copy(pb_hbm.at[pl.ds(base, _RPT)], pb_v)
    sems = (sem0, sem1)

    def fire(row, s):
        j = 2 * row
        pltpu.async_copy(tab_hbm.at[idx_v.at[j]], bufs.at[2 * s + 0], sems[s])
        pltpu.async_copy(tab_hbm.at[idx_v.at[j + 1]], bufs.at[2 * s + 1], sems[s])

    def drain(row, s):
        j = 2 * row
        pltpu.make_async_copy(tab_hbm.at[idx_v.at[j]], bufs.at[2 * s + 0], sems[s]).wait()
        pltpu.make_async_copy(tab_hbm.at[idx_v.at[j + 1]], bufs.at[2 * s + 1], sems[s]).wait()

    def reduce_row(row, s):
        pbvec = pb_v[row, pl.ds(0, _PW)]
        accs = tuple(jnp.zeros((16,), jnp.float32) for _ in range(4))
        for c01 in range(2):
            t0 = _HS * c01
            bslot = 2 * s + c01
            lo = t0
            while lo < t0 + _HS:
                widx = lo >> 5
                hi = min(32 * (widx + 1), t0 + _HS)
                word = pbvec[widx]

                def body(i, carry, word=word, t0=t0, bslot=bslot):
                    hb = ((word >> (i & 31)) & 1) << 6
                    return tuple(
                        carry[k] + bufs[bslot, i - t0, pl.ds(hb + 16 * k, 16)]
                        for k in range(4))

                accs = lax.fori_loop(lo, hi, body, accs)
                lo = hi
        for k in range(4):
            acc[row, pl.ds(16 * k, 16)] = accs[k]

    fire(0, 0)
    fire(1, 1)

    def outer(rr, carry):
        for s in range(2):
            row = 2 * rr + s
            drain(row, s)
            reduce_row(row, s)
            fire(row + 2, s)
        return carry

    lax.fori_loop(0, _RPT // 2 - 1, outer, 0)
    for s in range(2):
        drain(_RPT - 2 + s, s)
        reduce_row(_RPT - 2 + s, s)

    pltpu.sync_copy(acc, out_hbm.at[pl.ds(base, _RPT)])


@functools.partial(
    pl.kernel,
    mesh=plsc.VectorSubcoreMesh(core_axis_name="c", subcore_axis_name="s"),
    out_type=jax.ShapeDtypeStruct((_B, _D), jnp.float32),
    scratch_types=[
        pltpu.VMEM((2 * _RPT, _HS), jnp.int32),
        pltpu.VMEM((_RPT, _PW), jnp.int32),
        pltpu.VMEM((4, _HS, 2 * _D), jnp.float32),
        pltpu.VMEM((_RPT, _D), jnp.float32),
        pltpu.SemaphoreType.DMA,
        pltpu.SemaphoreType.DMA,
    ],
)
def _pool1(idx_hbm, pb_hbm, tab_hbm, out_hbm, idx_v, pb_v, bufs, acc, sem0, sem1):
    _pool_body(idx_hbm, pb_hbm, tab_hbm, out_hbm, idx_v, pb_v, bufs, acc, sem0, sem1)


_BM = 512


def _mlp_body(xw_ref, xn_ref, w1_ref, b1_ref, w2_ref, b2_ref, o_ref):
    x = jnp.concatenate([xw_ref[...], xn_ref[...]], axis=1) * (1.0 / _S)
    h = lax.dot_general(x, w1_ref[...], (((1,), (1,)), ((), ())),
                        preferred_element_type=jnp.float32)
    h = jnp.maximum(h + b1_ref[...], 0.0)
    o = lax.dot_general(h, w2_ref[...], (((1,), (1,)), ((), ())),
                        preferred_element_type=jnp.float32)
    o_ref[...] = o + b2_ref[...]


def _mlp(xw, xn, W1, b1, W2, b2):
    return pl.pallas_call(
        _mlp_body,
        grid=(_B // _BM,),
        in_specs=[
            pl.BlockSpec((_BM, _D), lambda i: (i, 0)),
            pl.BlockSpec((_BM, _D), lambda i: (i, 0)),
            pl.BlockSpec((_H, 2 * _D), lambda i: (0, 0)),
            pl.BlockSpec((1, _H), lambda i: (0, 0)),
            pl.BlockSpec((_C, _H), lambda i: (0, 0)),
            pl.BlockSpec((1, _C), lambda i: (0, 0)),
        ],
        out_specs=pl.BlockSpec((_BM, _C), lambda i: (i, 0)),
        out_shape=jax.ShapeDtypeStruct((_B, _C), jnp.float32),
    )(xw, xn, W1, b1.reshape(1, _H), W2, b2.reshape(1, _C))


def _packbits(hb):
    par = hb.astype(jnp.uint32)
    par = jnp.pad(par, ((0, 0), (0, 32 * _PW - _S)))
    par = par.reshape(_B, _PW, 32)
    shifts = jnp.arange(32, dtype=jnp.uint32)[None, None, :]
    return (par << shifts).sum(axis=-1).astype(jnp.int32)


def kernel(sequence, ngrams, word_emb, ngram_emb, W1, b1, W2, b2):
    seq_jh, seq_hb = _fold_idx(sequence.astype(jnp.int32))
    ng_jh, ng_hb = _fold_idx(ngrams.astype(jnp.int32))
    seqh = seq_jh.reshape(2 * _B, _HS)
    ngh = ng_jh.reshape(2 * _B, _HS)
    pbw = _packbits(seq_hb)
    pbn = _packbits(ng_hb)
    wtab = _transpose_table(word_emb.T, word_emb.shape[0])
    ntab = _transpose_table(ngram_emb.T, ngram_emb.shape[0])
    xw = _pool1(seqh, pbw, wtab)
    xn = _pool1(ngh, pbn, ntab)
    return _mlp(xw, xn, W1, b1, W2, b2)


# speedup vs baseline: 1.7797x; 1.1315x over previous
import functools

import jax
import jax.numpy as jnp
from jax import lax
from jax.experimental import pallas as pl
from jax.experimental.pallas import tpu as pltpu
from jax.experimental.pallas import tpu_sc as plsc

_B = 4096
_S = 200
_D = 64
_H = 256
_C = 10
_NC = 2
_NS = 16
_NW = _NC * _NS
_RPT = _B // _NW
_HS = _S // 2
_PW = 16

_TB = 16384


def _trans_body(in_ref, *rest):
    o_ref = rest[-1]
    xt = jnp.transpose(in_ref[...], (1, 0))
    o_ref[...] = jnp.concatenate(
        [xt[: _TB // 2], xt[_TB // 2:]], axis=1)


def _transpose_table(tabT, n_rows, after=None):
    grid = (n_rows + _TB - 1) // _TB
    in_specs = [pl.BlockSpec((_D, _TB), lambda i: (0, i))]
    args = [tabT]
    if after is not None:
        in_specs.append(pl.BlockSpec((1, 2 * _D), lambda i: (0, 0)))
        args.append(after[0:1])
    return pl.pallas_call(
        _trans_body,
        grid=(grid,),
        in_specs=in_specs,
        out_specs=pl.BlockSpec((_TB // 2, 2 * _D), lambda i: (i, 0)),
        out_shape=jax.ShapeDtypeStruct((grid * _TB // 2, 2 * _D), jnp.float32),
    )(*args)


def _fold_idx(idx):
    half = _TB // 2
    jh = (idx // _TB) * half + (idx % half)
    hb = (idx % _TB) // half
    return jh, hb


def _pool_body(idx_hbm, pb_hbm, tab_hbm, out_hbm, idx_v, pb_v, bufs, acc, sem0, sem1):
    wid = lax.axis_index("c") * _NS + lax.axis_index("s")
    base = wid * _RPT

    pltpu.sync_copy(idx_hbm.at[pl.ds(2 * base, 2 * _RPT)], idx_v)
    pltpu.sync_copy(pb_hbm.at[pl.ds(base, _RPT)], pb_v)
    sems = (sem0, sem1)

    def fire(row, s):
        j = 2 * row
        pltpu.async_copy(tab_hbm.at[idx_v.at[j]], bufs.at[2 * s + 0], sems[s])
        pltpu.async_copy(tab_hbm.at[idx_v.at[j + 1]], bufs.at[2 * s + 1], sems[s])

    def drain(row, s):
        j = 2 * row
        pltpu.make_async_copy(tab_hbm.at[idx_v.at[j]], bufs.at[2 * s + 0], sems[s]).wait()
        pltpu.make_async_copy(tab_hbm.at[idx_v.at[j + 1]], bufs.at[2 * s + 1], sems[s]).wait()

    def reduce_row(row, s):
        pbvec = pb_v[row, pl.ds(0, _PW)]
        accs = tuple(jnp.zeros((16,), jnp.float32) for _ in range(4))
        for c01 in range(2):
            t0 = _HS * c01
            bslot = 2 * s + c01
            lo = t0
            while lo < t0 + _HS:
                widx = lo >> 5
                hi = min(32 * (widx + 1), t0 + _HS)
                word = pbvec[widx]

                def body(i, carry, word=word, t0=t0, bslot=bslot):
                    hb = ((word >> (i & 31)) & 1) << 6
                    return tuple(
                        carry[k] + bufs[bslot, i - t0, pl.ds(hb + 16 * k, 16)]
                        for k in range(4))

                accs = lax.fori_loop(lo, hi, body, accs)
                lo = hi
        for k in range(4):
            acc[row, pl.ds(16 * k, 16)] = accs[k]

    fire(0, 0)
    fire(1, 1)

    def outer(rr, carry):
        for s in range(2):
            row = 2 * rr + s
            drain(row, s)
            reduce_row(row, s)
            fire(row + 2, s)
        return carry

    lax.fori_loop(0, _RPT // 2 - 1, outer, 0)
    for s in range(2):
        drain(_RPT - 2 + s, s)
        reduce_row(_RPT - 2 + s, s)

    pltpu.sync_copy(acc, out_hbm.at[pl.ds(base, _RPT)])


@functools.partial(
    pl.kernel,
    mesh=plsc.VectorSubcoreMesh(core_axis_name="c", subcore_axis_name="s"),
    out_type=jax.ShapeDtypeStruct((_B, _D), jnp.float32),
    scratch_types=[
        pltpu.VMEM((2 * _RPT, _HS), jnp.int32),
        pltpu.VMEM((_RPT, _PW), jnp.int32),
        pltpu.VMEM((4, _HS, 2 * _D), jnp.float32),
        pltpu.VMEM((_RPT, _D), jnp.float32),
        pltpu.SemaphoreType.DMA,
        pltpu.SemaphoreType.DMA,
    ],
)
def _pool1(idx_hbm, pb_hbm, tab_hbm, out_hbm, idx_v, pb_v, bufs, acc, sem0, sem1):
    _pool_body(idx_hbm, pb_hbm, tab_hbm, out_hbm, idx_v, pb_v, bufs, acc, sem0, sem1)


_BM = 512


def _mlp_body(xw_ref, xn_ref, w1_ref, b1_ref, w2_ref, b2_ref, o_ref):
    x = jnp.concatenate([xw_ref[...], xn_ref[...]], axis=1) * (1.0 / _S)
    h = lax.dot_general(x, w1_ref[...], (((1,), (1,)), ((), ())),
                        preferred_element_type=jnp.float32)
    h = jnp.maximum(h + b1_ref[...], 0.0)
    o = lax.dot_general(h, w2_ref[...], (((1,), (1,)), ((), ())),
                        preferred_element_type=jnp.float32)
    o_ref[...] = o + b2_ref[...]


def _mlp(xw, xn, W1, b1, W2, b2):
    return pl.pallas_call(
        _mlp_body,
        grid=(_B // _BM,),
        in_specs=[
            pl.BlockSpec((_BM, _D), lambda i: (i, 0)),
            pl.BlockSpec((_BM, _D), lambda i: (i, 0)),
            pl.BlockSpec((_H, 2 * _D), lambda i: (0, 0)),
            pl.BlockSpec((1, _H), lambda i: (0, 0)),
            pl.BlockSpec((_C, _H), lambda i: (0, 0)),
            pl.BlockSpec((1, _C), lambda i: (0, 0)),
        ],
        out_specs=pl.BlockSpec((_BM, _C), lambda i: (i, 0)),
        out_shape=jax.ShapeDtypeStruct((_B, _C), jnp.float32),
    )(xw, xn, W1, b1.reshape(1, _H), W2, b2.reshape(1, _C))


def _packbits(hb):
    par = hb.astype(jnp.uint32)
    par = jnp.pad(par, ((0, 0), (0, 32 * _PW - _S)))
    par = par.reshape(_B, _PW, 32)
    shifts = jnp.arange(32, dtype=jnp.uint32)[None, None, :]
    return (par << shifts).sum(axis=-1).astype(jnp.int32)


def kernel(sequence, ngrams, word_emb, ngram_emb, W1, b1, W2, b2):
    seq_jh, seq_hb = _fold_idx(sequence.astype(jnp.int32))
    ng_jh, ng_hb = _fold_idx(ngrams.astype(jnp.int32))
    seqh = seq_jh.reshape(2 * _B, _HS)
    ngh = ng_jh.reshape(2 * _B, _HS)
    pbw = _packbits(seq_hb)
    pbn = _packbits(ng_hb)
    wtab = _transpose_table(word_emb.T, word_emb.shape[0])
    ntab = _transpose_table(ngram_emb.T, ngram_emb.shape[0], after=wtab)
    xw = _pool1(seqh, pbw, wtab)
    xn = _pool1(ngh, pbn, ntab)
    return _mlp(xw, xn, W1, b1, W2, b2)
